# baseline (speedup 1.0000x reference)
import functools

import jax
import jax.numpy as jnp
from jax import lax
from jax.experimental import pallas as pl

NK = 64


def _fps_body(S, px_ref, py_ref, pz_ref, cx_ref, cy_ref, cz_ref):
    px = px_ref[...]
    py = py_ref[...]
    pz = pz_ref[...]
    B, N = px.shape
    iota = lax.broadcasted_iota(jnp.int32, (B, N), 1)
    iota_s = lax.broadcasted_iota(jnp.int32, (B, S), 1)
    x0 = px[:, 0:1]
    y0 = py[:, 0:1]
    z0 = pz[:, 0:1]
    dists = (px - x0) ** 2 + (py - y0) ** 2 + (pz - z0) ** 2
    zero_s = jnp.zeros((B, S), jnp.float32)
    cx = jnp.where(iota_s == 0, x0, zero_s)
    cy = jnp.where(iota_s == 0, y0, zero_s)
    cz = jnp.where(iota_s == 0, z0, zero_s)

    def body(i, st):
        cx, cy, cz, dists = st
        m = jnp.max(dists, axis=1, keepdims=True)
        first = jnp.min(jnp.where(dists >= m, iota, N), axis=1, keepdims=True)
        oh = iota == first
        lx = jnp.sum(jnp.where(oh, px, 0.0), axis=1, keepdims=True)
        ly = jnp.sum(jnp.where(oh, py, 0.0), axis=1, keepdims=True)
        lz = jnp.sum(jnp.where(oh, pz, 0.0), axis=1, keepdims=True)
        d = (px - lx) ** 2 + (py - ly) ** 2 + (pz - lz) ** 2
        dists = jnp.minimum(dists, d)
        sel = iota_s == i
        cx = jnp.where(sel, lx, cx)
        cy = jnp.where(sel, ly, cy)
        cz = jnp.where(sel, lz, cz)
        return cx, cy, cz, dists

    cx, cy, cz, _ = lax.fori_loop(1, S, body, (cx, cy, cz, dists))
    cx_ref[...] = cx
    cy_ref[...] = cy
    cz_ref[...] = cz


def _fps(px, py, pz, S):
    B, N = px.shape
    out = jax.ShapeDtypeStruct((B, S), jnp.float32)
    return pl.pallas_call(
        functools.partial(_fps_body, S),
        out_shape=(out, out, out),
    )(px, py, pz)


def _nbr_body(r2, nblk, pxc_ref, pyc_ref, pzc_ref,
              cx_ref, cy_ref, cz_ref, out_ref):
    p_x = pxc_ref[...][0, :, 0:1]
    p_y = pyc_ref[...][0, :, 0:1]
    p_z = pzc_ref[...][0, :, 0:1]
    c_x = cx_ref[...][0]
    c_y = cy_ref[...][0]
    c_z = cz_ref[...][0]
    N = p_x.shape[0]
    S = c_x.shape[1]
    d = (p_x - c_x) ** 2 + (p_y - c_y) ** 2 + (p_z - c_z) ** 2
    cnt = (d <= r2).astype(jnp.float32)
    off = 1
    while off < N:
        cnt = cnt + jnp.concatenate(
            [jnp.zeros((off, S), jnp.float32), cnt[: N - off]], axis=0)
        off *= 2
    rows = []
    for k in range(NK):
        rows.append(jnp.sum((cnt <= float(k)).astype(jnp.float32), axis=0,
                            keepdims=True))
    hs = S // nblk
    halves = []
    for h in range(nblk):
        halves.append(jnp.concatenate(
            [r[:, h * hs:(h + 1) * hs] for r in rows], axis=1))
    out = halves[0] if nblk == 1 else jnp.concatenate(halves, axis=0)
    out_ref[...] = out.astype(jnp.int32)[None]


def _nbr(pxc, pyc, pzc, cx2, cy2, cz2, r2, nblk):
    B, N, _ = pxc.shape
    S = cx2.shape[2]
    pts_spec = pl.BlockSpec((1, N, 8), lambda b: (b, 0, 0))
    cen_spec = pl.BlockSpec((1, 1, S), lambda b: (b, 0, 0))
    oshape = (B, nblk, NK * S // nblk)
    return pl.pallas_call(
        functools.partial(_nbr_body, r2, nblk),
        grid=(B,),
        in_specs=[pts_spec, pts_spec, pts_spec, cen_spec, cen_spec, cen_spec],
        out_specs=pl.BlockSpec((1,) + oshape[1:], lambda b: (b, 0, 0)),
        out_shape=jax.ShapeDtypeStruct(oshape, jnp.int32),
    )(pxc, pyc, pzc, cx2, cy2, cz2)


def _sa_body(N, sblk, *refs):
    (tab_ref, idx_ref, cen_ref, w1_ref, wr_ref, b1_ref, s1_ref, t1_ref,
     w2_ref, b2_ref, s2_ref, t2_ref, w3_ref, b3_ref, s3_ref, t3_ref,
     out_ref) = refs
    tab = tab_ref[...][0]
    idx = idx_ref[...][0]
    L = idx.shape[1]
    f1 = w1_ref.shape[0]
    pre = jnp.dot(w1_ref[...], tab, preferred_element_type=jnp.float32)
    idxc = jnp.minimum(idx, N - 1)
    idx2 = jnp.broadcast_to(idxc, (f1, L))
    h = jnp.zeros((f1, L), jnp.float32)
    for c in range(N // 128):
        part = pre[:, c * 128:(c + 1) * 128]
        li = jnp.clip(idx2 - c * 128, 0, 127)
        g = jnp.take_along_axis(part, li, axis=1)
        h = jnp.where(idx2 // 128 == c, g, h)
    ct = jnp.dot(wr_ref[...], cen_ref[...][0],
                 preferred_element_type=jnp.float32)
    ctr = jnp.concatenate([ct] * NK, axis=1)
    h = jnp.maximum(h - ctr + b1_ref[...], 0.0) * s1_ref[...] + t1_ref[...]
    h = jnp.dot(w2_ref[...], h, preferred_element_type=jnp.float32)
    h = jnp.maximum(h + b2_ref[...], 0.0) * s2_ref[...] + t2_ref[...]
    h = jnp.dot(w3_ref[...], h, preferred_element_type=jnp.float32)
    h = jnp.maximum(h + b3_ref[...], 0.0) * s3_ref[...] + t3_ref[...]
    h = h + (idx >= N).astype(jnp.float32) * (-1e30)
    m = h[:, 0:sblk]
    for c in range(1, NK):
        m = jnp.maximum(m, h[:, c * sblk:(c + 1) * sblk])
    out_ref[...] = m[None]


def _sa_fused(tabT, idxF, cenT, w1pT, wr8T, mats, S, nblk):
    B, dp, N = tabT.shape
    sblk = S // nblk
    f1 = w1pT.shape[0]
    f2 = mats["w2"].shape[1]
    f3 = mats["w3"].shape[1]
    vec = lambda f: pl.BlockSpec((f, 1), lambda b, j: (0, 0))
    mat = lambda a, b: pl.BlockSpec((a, b), lambda i, j: (0, 0))
    tv = lambda key: mats[key].reshape(-1, 1)
    return pl.pallas_call(
        functools.partial(_sa_body, N, sblk),
        grid=(B, nblk),
        in_specs=[
            pl.BlockSpec((1, dp, N), lambda b, j: (b, 0, 0)),
            pl.BlockSpec((1, 1, NK * sblk),
                         lambda b, j: (b * nblk + j, 0, 0)),
            pl.BlockSpec((1, 8, sblk), lambda b, j: (b, 0, j)),
            mat(f1, dp), mat(f1, 8), vec(f1), vec(f1), vec(f1),
            mat(f2, f1), vec(f2), vec(f2), vec(f2),
            mat(f3, f2), vec(f3), vec(f3), vec(f3),
        ],
        out_specs=pl.BlockSpec((1, f3, sblk), lambda b, j: (b, 0, j)),
        out_shape=jax.ShapeDtypeStruct((B, f3, S), jnp.float32),
    )(tabT, idxF.reshape(B * nblk, 1, NK * sblk),
      cenT, w1pT, wr8T, tv("b1"), tv("s1"), tv("t1"),
      mats["w2"].T, tv("b2"), tv("s2"), tv("t2"),
      mats["w3"].T, tv("b3"), tv("s3"), tv("t3"))


def _sa3_body(cb, npts, *refs):
    (rows_ref, cen_ref, w1a_ref, w1b_ref, b1_ref, s1_ref, t1_ref,
     w2_ref, b2_ref, s2_ref, t2_ref, w3_ref, b3_ref, s3_ref, t3_ref,
     out_ref) = refs
    h = (jnp.dot(rows_ref[...], w1a_ref[...],
                 preferred_element_type=jnp.float32)
         + jnp.dot(cen_ref[...], w1b_ref[...],
                   preferred_element_type=jnp.float32))
    h = jnp.maximum(h + b1_ref[...], 0.0) * s1_ref[...] + t1_ref[...]
    h = jnp.dot(h, w2_ref[...], preferred_element_type=jnp.float32)
    h = jnp.maximum(h + b2_ref[...], 0.0) * s2_ref[...] + t2_ref[...]
    h = jnp.dot(h, w3_ref[...], preferred_element_type=jnp.float32)
    h = jnp.maximum(h + b3_ref[...], 0.0) * s3_ref[...] + t3_ref[...]
    f3 = h.shape[1]
    out_ref[...] = jnp.max(jnp.reshape(h, (cb, npts, f3)), axis=1)


def _sa3(rows, cen, w1a, w1b, mats, npts, cb=8):
    SB = rows.shape[0]
    B = SB // npts
    fin = rows.shape[1]
    f1 = w1a.shape[1]
    f2 = mats["w2"].shape[1]
    f3 = mats["w3"].shape[1]
    vec = lambda f: pl.BlockSpec((1, f), lambda i: (0, 0))
    mat = lambda a, b: pl.BlockSpec((a, b), lambda i: (0, 0))
    body = lambda *refs: _sa3_body(cb, npts, *refs)
    return pl.pallas_call(
        body,
        grid=(B // cb,),
        in_specs=[
            pl.BlockSpec((cb * npts, fin), lambda i: (i, 0)),
            pl.BlockSpec((cb * npts, 8), lambda i: (i, 0)),
            mat(fin, f1), mat(8, f1), vec(f1), vec(f1), vec(f1),
            mat(f1, f2), vec(f2), vec(f2), vec(f2),
            mat(f2, f3), vec(f3), vec(f3), vec(f3),
        ],
        out_specs=pl.BlockSpec((cb, f3), lambda i: (i, 0)),
        out_shape=jax.ShapeDtypeStruct((B, f3), jnp.float32),
    )(rows, cen, w1a, w1b, mats["b1"], mats["s1"], mats["t1"],
      mats["w2"], mats["b2"], mats["s2"], mats["t2"],
      mats["w3"], mats["b3"], mats["s3"], mats["t3"])


def _head_body(nseq, nstep, *refs):
    (g_ref, w1_ref, b1_ref, w2_ref, b2_ref, w3_ref, b3_ref,
     wih_ref, bih_ref, whh_ref, bhh_ref, w4_ref, b4_ref, w5_ref, b5_ref,
     out_ref) = refs
    h = jnp.maximum(jnp.dot(g_ref[...], w1_ref[...],
                            preferred_element_type=jnp.float32)
                    + b1_ref[...], 0.0)
    h = jnp.maximum(jnp.dot(h, w2_ref[...],
                            preferred_element_type=jnp.float32)
                    + b2_ref[...], 0.0)
    h = jnp.dot(h, w3_ref[...], preferred_element_type=jnp.float32) \
        + b3_ref[...]
    hid = h.shape[1]
    hs = jnp.zeros((nseq, hid), jnp.float32)
    for t in range(nstep):
        xt = h[t * nseq:(t + 1) * nseq, :]
        gi = jnp.dot(xt, wih_ref[...],
                     preferred_element_type=jnp.float32) + bih_ref[...]
        gh = jnp.dot(hs, whh_ref[...],
                     preferred_element_type=jnp.float32) + bhh_ref[...]
        i_r = gi[:, 0 * hid:1 * hid]
        i_z = gi[:, 1 * hid:2 * hid]
        i_n = gi[:, 2 * hid:3 * hid]
        h_r = gh[:, 0 * hid:1 * hid]
        h_z = gh[:, 1 * hid:2 * hid]
        h_n = gh[:, 2 * hid:3 * hid]
        rg = jax.nn.sigmoid(i_r + h_r)
        zg = jax.nn.sigmoid(i_z + h_z)
        ng = jnp.tanh(i_n + rg * h_n)
        hs = (1.0 - zg) * ng + zg * hs
    h = jnp.maximum(jnp.dot(hs, w4_ref[...],
                            preferred_element_type=jnp.float32)
                    + b4_ref[...], 0.0)
    out_ref[...] = jnp.dot(h, w5_ref[...],
                           preferred_element_type=jnp.float32) + b5_ref[...]


def _head(g, p, nseq, nstep):
    body = lambda *refs: _head_body(nseq, nstep, *refs)
    zdim = p["lin5"]["W"].shape[1]
    return pl.pallas_call(
        body,
        out_shape=jax.ShapeDtypeStruct((nseq, zdim), jnp.float32),
    )(g, p["lin1"]["W"], p["lin1"]["b"].reshape(1, -1),
      p["lin2"]["W"], p["lin2"]["b"].reshape(1, -1),
      p["lin3"]["W"], p["lin3"]["b"].reshape(1, -1),
      p["gru"]["W_ih"], p["gru"]["b_ih"].reshape(1, -1),
      p["gru"]["W_hh"], p["gru"]["b_hh"].reshape(1, -1),
      p["lin4"]["W"], p["lin4"]["b"].reshape(1, -1),
      p["lin5"]["W"], p["lin5"]["b"].reshape(1, -1))


_BN_SCALE = float((1.0 + 1e-5) ** -0.5)


def _sa_params(layers, fx, dp):
    w1 = layers[0]["W"]
    f1 = w1.shape[1]
    w1p = jnp.zeros((dp, f1), jnp.float32)
    w1p = w1p.at[:fx].set(w1[:fx]).at[fx:fx + 3].set(w1[fx:fx + 3])
    wr8 = jnp.zeros((8, f1), jnp.float32).at[:3].set(w1[fx:fx + 3])
    mats = {"b1": layers[0]["b"].reshape(1, -1),
            "s1": (layers[0]["gamma"] * _BN_SCALE).reshape(1, -1),
            "t1": layers[0]["beta"].reshape(1, -1)}
    for i, nm in ((1, "2"), (2, "3")):
        mats["w" + nm] = layers[i]["W"]
        mats["b" + nm] = layers[i]["b"].reshape(1, -1)
        mats["s" + nm] = (layers[i]["gamma"] * _BN_SCALE).reshape(1, -1)
        mats["t" + nm] = layers[i]["beta"].reshape(1, -1)
    return w1p, wr8, mats


def _cen8(cx, cy, cz):
    B, S = cx.shape
    c = jnp.stack([cx, cy, cz], axis=-1).reshape(B * S, 3)
    return jnp.concatenate([c, jnp.zeros((B * S, 5), jnp.float32)], axis=1)


def _expand_pts(p):
    return jnp.broadcast_to(p[:, :, None], (*p.shape, 8))


def kernel(x, pos, batch, params):
    B = 64
    N = 1024
    S1, S2 = 512, 128
    pr = pos.reshape(B, N, 3)
    px, py, pz = pr[:, :, 0], pr[:, :, 1], pr[:, :, 2]

    c1x, c1y, c1z = _fps(px, py, pz, S1)
    idxF1 = _nbr(_expand_pts(px), _expand_pts(py), _expand_pts(pz),
                 c1x.reshape(B, 1, S1), c1y.reshape(B, 1, S1),
                 c1z.reshape(B, 1, S1), 0.2 * 0.2, nblk=2)
    xT = x.reshape(B, N, 9).transpose(0, 2, 1)
    pT = pr.transpose(0, 2, 1)
    tabT1 = jnp.concatenate(
        [xT, pT, jnp.zeros((B, 4, N), jnp.float32)], axis=1)
    cenT1 = jnp.concatenate(
        [jnp.stack([c1x, c1y, c1z], axis=1),
         jnp.zeros((B, 5, S1), jnp.float32)], axis=1)
    w1p, wr8, mats = _sa_params(params["sa1"], 9, 16)
    h1T = _sa_fused(tabT1, idxF1, cenT1, w1p.T, wr8.T, mats, S1,
                    nblk=2)

    c2x, c2y, c2z = _fps(c1x, c1y, c1z, S2)
    idxF2 = _nbr(_expand_pts(c1x), _expand_pts(c1y), _expand_pts(c1z),
                 c2x.reshape(B, 1, S2), c2y.reshape(B, 1, S2),
                 c2z.reshape(B, 1, S2), 0.4 * 0.4, nblk=1)
    tabT2 = jnp.concatenate(
        [h1T, cenT1[:, :3], jnp.zeros((B, 5, S1), jnp.float32)],
        axis=1)
    cenT2 = jnp.concatenate(
        [jnp.stack([c2x, c2y, c2z], axis=1),
         jnp.zeros((B, 5, S2), jnp.float32)], axis=1)
    w1p2, wr82, mats2 = _sa_params(params["sa2"], 128, 136)
    h2T = _sa_fused(tabT2, idxF2, cenT2, w1p2.T, wr82.T, mats2, S2,
                    nblk=1)
    h2 = h2T.transpose(0, 2, 1).reshape(B * S2, -1)
    cen2 = _cen8(c2x, c2y, c2z)

    w1 = params["sa3"][0]["W"]
    w1a = w1[:256]
    w1b = jnp.zeros((8, 256), jnp.float32).at[:3].set(w1[256:259])
    mats3 = {"b1": params["sa3"][0]["b"].reshape(1, -1),
             "s1": (params["sa3"][0]["gamma"] * _BN_SCALE).reshape(1, -1),
             "t1": params["sa3"][0]["beta"].reshape(1, -1)}
    for i, nm in ((1, "2"), (2, "3")):
        mats3["w" + nm] = params["sa3"][i]["W"]
        mats3["b" + nm] = params["sa3"][i]["b"].reshape(1, -1)
        mats3["s" + nm] = (params["sa3"][i]["gamma"] * _BN_SCALE).reshape(1, -1)
        mats3["t" + nm] = params["sa3"][i]["beta"].reshape(1, -1)
    g = _sa3(h2, cen2, w1a, w1b, mats3, npts=S2)

    nseq, nstep = B // 4, 4
    g_r = g.reshape(nseq, nstep, -1).transpose(1, 0, 2).reshape(B, -1)
    return _head(g_r, params, nseq, nstep)


# speedup vs baseline: 1.8108x; 1.8108x over previous
import functools

import jax
import jax.numpy as jnp
from jax import lax
from jax.experimental import pallas as pl

NK = 64


def _fps_body(S, px_ref, py_ref, pz_ref, cx_ref, cy_ref, cz_ref):
    px = px_ref[...]
    py = py_ref[...]
    pz = pz_ref[...]
    B, N = px.shape
    iota = lax.broadcasted_iota(jnp.int32, (B, N), 1)
    iota_s = lax.broadcasted_iota(jnp.int32, (B, S), 1)
    x0 = px[:, 0:1]
    y0 = py[:, 0:1]
    z0 = pz[:, 0:1]
    dists = (px - x0) ** 2 + (py - y0) ** 2 + (pz - z0) ** 2
    zero_s = jnp.zeros((B, S), jnp.float32)
    cx = jnp.where(iota_s == 0, x0, zero_s)
    cy = jnp.where(iota_s == 0, y0, zero_s)
    cz = jnp.where(iota_s == 0, z0, zero_s)

    def body(i, st):
        cx, cy, cz, dists = st
        m = jnp.max(dists, axis=1, keepdims=True)
        first = jnp.min(jnp.where(dists >= m, iota, N), axis=1, keepdims=True)
        oh = iota == first
        lx = jnp.sum(jnp.where(oh, px, 0.0), axis=1, keepdims=True)
        ly = jnp.sum(jnp.where(oh, py, 0.0), axis=1, keepdims=True)
        lz = jnp.sum(jnp.where(oh, pz, 0.0), axis=1, keepdims=True)
        d = (px - lx) ** 2 + (py - ly) ** 2 + (pz - lz) ** 2
        dists = jnp.minimum(dists, d)
        sel = iota_s == i
        cx = jnp.where(sel, lx, cx)
        cy = jnp.where(sel, ly, cy)
        cz = jnp.where(sel, lz, cz)
        return cx, cy, cz, dists

    cx, cy, cz, _ = lax.fori_loop(1, S, body, (cx, cy, cz, dists))
    cx_ref[...] = cx
    cy_ref[...] = cy
    cz_ref[...] = cz


def _fps(px, py, pz, S):
    B, N = px.shape
    out = jax.ShapeDtypeStruct((B, S), jnp.float32)
    return pl.pallas_call(
        functools.partial(_fps_body, S),
        out_shape=(out, out, out),
    )(px, py, pz)


def _nbr_body(r2, nblk, pxc_ref, pyc_ref, pzc_ref,
              cx_ref, cy_ref, cz_ref, out_ref):
    p_x = pxc_ref[...][0, :, 0:1]
    p_y = pyc_ref[...][0, :, 0:1]
    p_z = pzc_ref[...][0, :, 0:1]
    c_x = cx_ref[...][0]
    c_y = cy_ref[...][0]
    c_z = cz_ref[...][0]
    N = p_x.shape[0]
    S = c_x.shape[1]
    d = (p_x - c_x) ** 2 + (p_y - c_y) ** 2 + (p_z - c_z) ** 2
    cnt = (d <= r2).astype(jnp.float32)
    off = 1
    while off < N:
        cnt = cnt + jnp.concatenate(
            [jnp.zeros((off, S), jnp.float32), cnt[: N - off]], axis=0)
        off *= 2
    rows = []
    for k in range(NK):
        rows.append(jnp.sum((cnt <= float(k)).astype(jnp.float32), axis=0,
                            keepdims=True))
    hs = S // nblk
    halves = []
    for h in range(nblk):
        halves.append(jnp.concatenate(
            [r[:, h * hs:(h + 1) * hs] for r in rows], axis=1))
    out = halves[0] if nblk == 1 else jnp.concatenate(halves, axis=0)
    out_ref[...] = out.astype(jnp.int32)[None]


def _nbr(pxc, pyc, pzc, cx2, cy2, cz2, r2, nblk):
    B, N, _ = pxc.shape
    S = cx2.shape[2]
    pts_spec = pl.BlockSpec((1, N, 8), lambda b: (b, 0, 0))
    cen_spec = pl.BlockSpec((1, 1, S), lambda b: (b, 0, 0))
    oshape = (B, nblk, NK * S // nblk)
    return pl.pallas_call(
        functools.partial(_nbr_body, r2, nblk),
        grid=(B,),
        in_specs=[pts_spec, pts_spec, pts_spec, cen_spec, cen_spec, cen_spec],
        out_specs=pl.BlockSpec((1,) + oshape[1:], lambda b: (b, 0, 0)),
        out_shape=jax.ShapeDtypeStruct(oshape, jnp.int32),
    )(pxc, pyc, pzc, cx2, cy2, cz2)


def _sa_body(N, sblk, *refs):
    (tab_ref, idx_ref, cen_ref, w1_ref, wr_ref, b1_ref, s1_ref, t1_ref,
     w2_ref, b2_ref, s2_ref, t2_ref, w3_ref, b3_ref, s3_ref, t3_ref,
     out_ref) = refs
    tab = tab_ref[...][0]
    idx = idx_ref[...][0]
    L = idx.shape[1]
    dp = tab.shape[0]
    idxc = jnp.minimum(idx, N - 1)
    idx2 = jnp.broadcast_to(idxc, (dp, L))
    rows = jnp.zeros((dp, L), jnp.float32)
    for c in range(N // 128):
        part = tab[:, c * 128:(c + 1) * 128]
        li = jnp.clip(idx2 - c * 128, 0, 127)
        g = jnp.take_along_axis(part, li, axis=1)
        rows = jnp.where(idx2 // 128 == c, g, rows)
    h = jnp.dot(w1_ref[...], rows, preferred_element_type=jnp.float32)
    ct = jnp.dot(wr_ref[...], cen_ref[...][0],
                 preferred_element_type=jnp.float32)
    ctr = jnp.concatenate([ct] * NK, axis=1)
    h = jnp.maximum(h - ctr + b1_ref[...], 0.0) * s1_ref[...] + t1_ref[...]
    h = jnp.dot(w2_ref[...], h, preferred_element_type=jnp.float32)
    h = jnp.maximum(h + b2_ref[...], 0.0) * s2_ref[...] + t2_ref[...]
    h = jnp.dot(w3_ref[...], h, preferred_element_type=jnp.float32)
    h = jnp.maximum(h + b3_ref[...], 0.0) * s3_ref[...] + t3_ref[...]
    h = h + (idx >= N).astype(jnp.float32) * (-1e30)
    m = h[:, 0:sblk]
    for c in range(1, NK):
        m = jnp.maximum(m, h[:, c * sblk:(c + 1) * sblk])
    out_ref[...] = m[None]


def _sa_fused(tabT, idxF, cenT, w1pT, wr8T, mats, S, nblk):
    B, dp, N = tabT.shape
    sblk = S // nblk
    f1 = w1pT.shape[0]
    f2 = mats["w2"].shape[1]
    f3 = mats["w3"].shape[1]
    vec = lambda f: pl.BlockSpec((f, 1), lambda b, j: (0, 0))
    mat = lambda a, b: pl.BlockSpec((a, b), lambda i, j: (0, 0))
    tv = lambda key: mats[key].reshape(-1, 1)
    return pl.pallas_call(
        functools.partial(_sa_body, N, sblk),
        grid=(B, nblk),
        in_specs=[
            pl.BlockSpec((1, dp, N), lambda b, j: (b, 0, 0)),
            pl.BlockSpec((1, 1, NK * sblk),
                         lambda b, j: (b * nblk + j, 0, 0)),
            pl.BlockSpec((1, 8, sblk), lambda b, j: (b, 0, j)),
            mat(f1, dp), mat(f1, 8), vec(f1), vec(f1), vec(f1),
            mat(f2, f1), vec(f2), vec(f2), vec(f2),
            mat(f3, f2), vec(f3), vec(f3), vec(f3),
        ],
        out_specs=pl.BlockSpec((1, f3, sblk), lambda b, j: (b, 0, j)),
        out_shape=jax.ShapeDtypeStruct((B, f3, S), jnp.float32),
    )(tabT, idxF.reshape(B * nblk, 1, NK * sblk),
      cenT, w1pT, wr8T, tv("b1"), tv("s1"), tv("t1"),
      mats["w2"].T, tv("b2"), tv("s2"), tv("t2"),
      mats["w3"].T, tv("b3"), tv("s3"), tv("t3"))


def _sa3_body(cb, npts, *refs):
    (rows_ref, cen_ref, w1a_ref, w1b_ref, b1_ref, s1_ref, t1_ref,
     w2_ref, b2_ref, s2_ref, t2_ref, w3_ref, b3_ref, s3_ref, t3_ref,
     out_ref) = refs
    h = (jnp.dot(rows_ref[...], w1a_ref[...],
                 preferred_element_type=jnp.float32)
         + jnp.dot(cen_ref[...], w1b_ref[...],
                   preferred_element_type=jnp.float32))
    h = jnp.maximum(h + b1_ref[...], 0.0) * s1_ref[...] + t1_ref[...]
    h = jnp.dot(h, w2_ref[...], preferred_element_type=jnp.float32)
    h = jnp.maximum(h + b2_ref[...], 0.0) * s2_ref[...] + t2_ref[...]
    h = jnp.dot(h, w3_ref[...], preferred_element_type=jnp.float32)
    h = jnp.maximum(h + b3_ref[...], 0.0) * s3_ref[...] + t3_ref[...]
    f3 = h.shape[1]
    out_ref[...] = jnp.max(jnp.reshape(h, (cb, npts, f3)), axis=1)


def _sa3(rows, cen, w1a, w1b, mats, npts, cb=8):
    SB = rows.shape[0]
    B = SB // npts
    fin = rows.shape[1]
    f1 = w1a.shape[1]
    f2 = mats["w2"].shape[1]
    f3 = mats["w3"].shape[1]
    vec = lambda f: pl.BlockSpec((1, f), lambda i: (0, 0))
    mat = lambda a, b: pl.BlockSpec((a, b), lambda i: (0, 0))
    body = lambda *refs: _sa3_body(cb, npts, *refs)
    return pl.pallas_call(
        body,
        grid=(B // cb,),
        in_specs=[
            pl.BlockSpec((cb * npts, fin), lambda i: (i, 0)),
            pl.BlockSpec((cb * npts, 8), lambda i: (i, 0)),
            mat(fin, f1), mat(8, f1), vec(f1), vec(f1), vec(f1),
            mat(f1, f2), vec(f2), vec(f2), vec(f2),
            mat(f2, f3), vec(f3), vec(f3), vec(f3),
        ],
        out_specs=pl.BlockSpec((cb, f3), lambda i: (i, 0)),
        out_shape=jax.ShapeDtypeStruct((B, f3), jnp.float32),
    )(rows, cen, w1a, w1b, mats["b1"], mats["s1"], mats["t1"],
      mats["w2"], mats["b2"], mats["s2"], mats["t2"],
      mats["w3"], mats["b3"], mats["s3"], mats["t3"])


def _head_body(nseq, nstep, *refs):
    (g_ref, w1_ref, b1_ref, w2_ref, b2_ref, w3_ref, b3_ref,
     wih_ref, bih_ref, whh_ref, bhh_ref, w4_ref, b4_ref, w5_ref, b5_ref,
     out_ref) = refs
    h = jnp.maximum(jnp.dot(g_ref[...], w1_ref[...],
                            preferred_element_type=jnp.float32)
                    + b1_ref[...], 0.0)
    h = jnp.maximum(jnp.dot(h, w2_ref[...],
                            preferred_element_type=jnp.float32)
                    + b2_ref[...], 0.0)
    h = jnp.dot(h, w3_ref[...], preferred_element_type=jnp.float32) \
        + b3_ref[...]
    hid = h.shape[1]
    hs = jnp.zeros((nseq, hid), jnp.float32)
    for t in range(nstep):
        xt = h[t * nseq:(t + 1) * nseq, :]
        gi = jnp.dot(xt, wih_ref[...],
                     preferred_element_type=jnp.float32) + bih_ref[...]
        gh = jnp.dot(hs, whh_ref[...],
                     preferred_element_type=jnp.float32) + bhh_ref[...]
        i_r = gi[:, 0 * hid:1 * hid]
        i_z = gi[:, 1 * hid:2 * hid]
        i_n = gi[:, 2 * hid:3 * hid]
        h_r = gh[:, 0 * hid:1 * hid]
        h_z = gh[:, 1 * hid:2 * hid]
        h_n = gh[:, 2 * hid:3 * hid]
        rg = jax.nn.sigmoid(i_r + h_r)
        zg = jax.nn.sigmoid(i_z + h_z)
        ng = jnp.tanh(i_n + rg * h_n)
        hs = (1.0 - zg) * ng + zg * hs
    h = jnp.maximum(jnp.dot(hs, w4_ref[...],
                            preferred_element_type=jnp.float32)
                    + b4_ref[...], 0.0)
    out_ref[...] = jnp.dot(h, w5_ref[...],
                           preferred_element_type=jnp.float32) + b5_ref[...]


def _head(g, p, nseq, nstep):
    body = lambda *refs: _head_body(nseq, nstep, *refs)
    zdim = p["lin5"]["W"].shape[1]
    return pl.pallas_call(
        body,
        out_shape=jax.ShapeDtypeStruct((nseq, zdim), jnp.float32),
    )(g, p["lin1"]["W"], p["lin1"]["b"].reshape(1, -1),
      p["lin2"]["W"], p["lin2"]["b"].reshape(1, -1),
      p["lin3"]["W"], p["lin3"]["b"].reshape(1, -1),
      p["gru"]["W_ih"], p["gru"]["b_ih"].reshape(1, -1),
      p["gru"]["W_hh"], p["gru"]["b_hh"].reshape(1, -1),
      p["lin4"]["W"], p["lin4"]["b"].reshape(1, -1),
      p["lin5"]["W"], p["lin5"]["b"].reshape(1, -1))


_BN_SCALE = float((1.0 + 1e-5) ** -0.5)


def _sa_params(layers, fx, dp):
    w1 = layers[0]["W"]
    f1 = w1.shape[1]
    w1p = jnp.zeros((dp, f1), jnp.float32)
    w1p = w1p.at[:fx].set(w1[:fx]).at[fx:fx + 3].set(w1[fx:fx + 3])
    wr8 = jnp.zeros((8, f1), jnp.float32).at[:3].set(w1[fx:fx + 3])
    mats = {"b1": layers[0]["b"].reshape(1, -1),
            "s1": (layers[0]["gamma"] * _BN_SCALE).reshape(1, -1),
            "t1": layers[0]["beta"].reshape(1, -1)}
    for i, nm in ((1, "2"), (2, "3")):
        mats["w" + nm] = layers[i]["W"]
        mats["b" + nm] = layers[i]["b"].reshape(1, -1)
        mats["s" + nm] = (layers[i]["gamma"] * _BN_SCALE).reshape(1, -1)
        mats["t" + nm] = layers[i]["beta"].reshape(1, -1)
    return w1p, wr8, mats


def _cen8(cx, cy, cz):
    B, S = cx.shape
    c = jnp.stack([cx, cy, cz], axis=-1).reshape(B * S, 3)
    return jnp.concatenate([c, jnp.zeros((B * S, 5), jnp.float32)], axis=1)


def _expand_pts(p):
    return jnp.broadcast_to(p[:, :, None], (*p.shape, 8))


def kernel(x, pos, batch, params):
    B = 64
    N = 1024
    S1, S2 = 512, 128
    pr = pos.reshape(B, N, 3)
    px, py, pz = pr[:, :, 0], pr[:, :, 1], pr[:, :, 2]

    c1x, c1y, c1z = _fps(px, py, pz, S1)
    idxF1 = _nbr(_expand_pts(px), _expand_pts(py), _expand_pts(pz),
                 c1x.reshape(B, 1, S1), c1y.reshape(B, 1, S1),
                 c1z.reshape(B, 1, S1), 0.2 * 0.2, nblk=2)
    xT = x.reshape(B, N, 9).transpose(0, 2, 1)
    pT = pr.transpose(0, 2, 1)
    tabT1 = jnp.concatenate(
        [xT, pT, jnp.zeros((B, 4, N), jnp.float32)], axis=1)
    cenT1 = jnp.concatenate(
        [jnp.stack([c1x, c1y, c1z], axis=1),
         jnp.zeros((B, 5, S1), jnp.float32)], axis=1)
    w1p, wr8, mats = _sa_params(params["sa1"], 9, 16)
    h1T = _sa_fused(tabT1, idxF1, cenT1, w1p.T, wr8.T, mats, S1,
                    nblk=2)

    c2x, c2y, c2z = _fps(c1x, c1y, c1z, S2)
    idxF2 = _nbr(_expand_pts(c1x), _expand_pts(c1y), _expand_pts(c1z),
                 c2x.reshape(B, 1, S2), c2y.reshape(B, 1, S2),
                 c2z.reshape(B, 1, S2), 0.4 * 0.4, nblk=1)
    tabT2 = jnp.concatenate(
        [h1T, cenT1[:, :3], jnp.zeros((B, 5, S1), jnp.float32)],
        axis=1)
    cenT2 = jnp.concatenate(
        [jnp.stack([c2x, c2y, c2z], axis=1),
         jnp.zeros((B, 5, S2), jnp.float32)], axis=1)
    w1p2, wr82, mats2 = _sa_params(params["sa2"], 128, 136)
    h2T = _sa_fused(tabT2, idxF2, cenT2, w1p2.T, wr82.T, mats2, S2,
                    nblk=1)
    h2 = h2T.transpose(0, 2, 1).reshape(B * S2, -1)
    cen2 = _cen8(c2x, c2y, c2z)

    w1 = params["sa3"][0]["W"]
    w1a = w1[:256]
    w1b = jnp.zeros((8, 256), jnp.float32).at[:3].set(w1[256:259])
    mats3 = {"b1": params["sa3"][0]["b"].reshape(1, -1),
             "s1": (params["sa3"][0]["gamma"] * _BN_SCALE).reshape(1, -1),
             "t1": params["sa3"][0]["beta"].reshape(1, -1)}
    for i, nm in ((1, "2"), (2, "3")):
        mats3["w" + nm] = params["sa3"][i]["W"]
        mats3["b" + nm] = params["sa3"][i]["b"].reshape(1, -1)
        mats3["s" + nm] = (params["sa3"][i]["gamma"] * _BN_SCALE).reshape(1, -1)
        mats3["t" + nm] = params["sa3"][i]["beta"].reshape(1, -1)
    g = _sa3(h2, cen2, w1a, w1b, mats3, npts=S2)

    nseq, nstep = B // 4, 4
    g_r = g.reshape(nseq, nstep, -1).transpose(1, 0, 2).reshape(B, -1)
    return _head(g_r, params, nseq, nstep)


# speedup vs baseline: 1.8296x; 1.0104x over previous
import functools

import jax
import jax.numpy as jnp
from jax import lax
from jax.experimental import pallas as pl

NK = 64


def _fps_body(S, px_ref, py_ref, pz_ref, cx_ref, cy_ref, cz_ref):
    px = px_ref[...]
    py = py_ref[...]
    pz = pz_ref[...]
    B, N = px.shape
    iota = lax.broadcasted_iota(jnp.int32, (B, N), 1)
    iota_s = lax.broadcasted_iota(jnp.int32, (B, S), 1)
    x0 = px[:, 0:1]
    y0 = py[:, 0:1]
    z0 = pz[:, 0:1]
    dists = (px - x0) ** 2 + (py - y0) ** 2 + (pz - z0) ** 2
    zero_s = jnp.zeros((B, S), jnp.float32)
    cx = jnp.where(iota_s == 0, x0, zero_s)
    cy = jnp.where(iota_s == 0, y0, zero_s)
    cz = jnp.where(iota_s == 0, z0, zero_s)

    def body(i, st):
        cx, cy, cz, dists = st
        m = jnp.max(dists, axis=1, keepdims=True)
        first = jnp.min(jnp.where(dists >= m, iota, N), axis=1, keepdims=True)
        oh = iota == first
        lx = jnp.sum(jnp.where(oh, px, 0.0), axis=1, keepdims=True)
        ly = jnp.sum(jnp.where(oh, py, 0.0), axis=1, keepdims=True)
        lz = jnp.sum(jnp.where(oh, pz, 0.0), axis=1, keepdims=True)
        d = (px - lx) ** 2 + (py - ly) ** 2 + (pz - lz) ** 2
        dists = jnp.minimum(dists, d)
        sel = iota_s == i
        cx = jnp.where(sel, lx, cx)
        cy = jnp.where(sel, ly, cy)
        cz = jnp.where(sel, lz, cz)
        return cx, cy, cz, dists

    cx, cy, cz, _ = lax.fori_loop(1, S, body, (cx, cy, cz, dists))
    cx_ref[...] = cx
    cy_ref[...] = cy
    cz_ref[...] = cz


def _fps(px, py, pz, S):
    B, N = px.shape
    out = jax.ShapeDtypeStruct((B, S), jnp.float32)
    return pl.pallas_call(
        functools.partial(_fps_body, S),
        out_shape=(out, out, out),
    )(px, py, pz)


def _nbr_body(r2, nblk, pxc_ref, pyc_ref, pzc_ref,
              cx_ref, cy_ref, cz_ref, out_ref):
    p_x = pxc_ref[...][0, :, 0:1]
    p_y = pyc_ref[...][0, :, 0:1]
    p_z = pzc_ref[...][0, :, 0:1]
    c_x = cx_ref[...][0]
    c_y = cy_ref[...][0]
    c_z = cz_ref[...][0]
    N = p_x.shape[0]
    S = c_x.shape[1]
    d = (p_x - c_x) ** 2 + (p_y - c_y) ** 2 + (p_z - c_z) ** 2
    cnt = (d <= r2).astype(jnp.float32)
    off = 1
    while off < N:
        cnt = cnt + jnp.concatenate(
            [jnp.zeros((off, S), jnp.float32), cnt[: N - off]], axis=0)
        off *= 2
    rows = []
    for k in range(NK):
        rows.append(jnp.sum((cnt <= float(k)).astype(jnp.float32), axis=0,
                            keepdims=True))
    hs = S // nblk
    halves = []
    for h in range(nblk):
        halves.append(jnp.concatenate(
            [r[:, h * hs:(h + 1) * hs] for r in rows], axis=1))
    out = halves[0] if nblk == 1 else jnp.concatenate(halves, axis=0)
    out_ref[...] = out.astype(jnp.int32)[None]


def _nbr(pxc, pyc, pzc, cx2, cy2, cz2, r2, nblk):
    B, N, _ = pxc.shape
    S = cx2.shape[2]
    pts_spec = pl.BlockSpec((1, N, 8), lambda b: (b, 0, 0))
    cen_spec = pl.BlockSpec((1, 1, S), lambda b: (b, 0, 0))
    oshape = (B, nblk, NK * S // nblk)
    return pl.pallas_call(
        functools.partial(_nbr_body, r2, nblk),
        grid=(B,),
        in_specs=[pts_spec, pts_spec, pts_spec, cen_spec, cen_spec, cen_spec],
        out_specs=pl.BlockSpec((1,) + oshape[1:], lambda b: (b, 0, 0)),
        out_shape=jax.ShapeDtypeStruct(oshape, jnp.int32),
    )(pxc, pyc, pzc, cx2, cy2, cz2)


def _sa_body(N, sblk, *refs):
    (tab_ref, idx_ref, cen_ref, w1_ref, wr_ref, b1_ref, s1_ref, t1_ref,
     w2_ref, b2_ref, s2_ref, t2_ref, w3_ref, b3_ref, s3_ref, t3_ref,
     out_ref) = refs
    tab = tab_ref[...][0]
    idx = idx_ref[...][0]
    L = idx.shape[1]
    dp = tab.shape[0]
    idxc = jnp.minimum(idx, N - 1)
    idx2 = jnp.broadcast_to(idxc, (dp, L))
    rows = jnp.zeros((dp, L), jnp.float32)
    for c in range(N // 128):
        part = tab[:, c * 128:(c + 1) * 128]
        li = jnp.clip(idx2 - c * 128, 0, 127)
        g = jnp.take_along_axis(part, li, axis=1)
        rows = jnp.where(idx2 // 128 == c, g, rows)
    h = jnp.dot(w1_ref[...], rows, preferred_element_type=jnp.float32)
    ct = jnp.dot(wr_ref[...], cen_ref[...][0],
                 preferred_element_type=jnp.float32)
    ctr = jnp.concatenate([ct] * NK, axis=1)
    h = jnp.maximum(h - ctr + b1_ref[...], 0.0) * s1_ref[...] + t1_ref[...]
    h = jnp.dot(w2_ref[...], h, preferred_element_type=jnp.float32)
    h = jnp.maximum(h + b2_ref[...], 0.0) * s2_ref[...] + t2_ref[...]
    h = jnp.dot(w3_ref[...], h, preferred_element_type=jnp.float32)
    h = jnp.maximum(h + b3_ref[...], 0.0) * s3_ref[...] + t3_ref[...]
    h = h + (idx >= N).astype(jnp.float32) * (-1e30)
    m = h[:, 0:sblk]
    for c in range(1, NK):
        m = jnp.maximum(m, h[:, c * sblk:(c + 1) * sblk])
    out_ref[...] = m[None]


def _sa_fused(tabT, idxF, cenT, w1pT, wr8T, mats, S, nblk):
    B, dp, N = tabT.shape
    sblk = S // nblk
    f1 = w1pT.shape[0]
    f2 = mats["w2"].shape[1]
    f3 = mats["w3"].shape[1]
    vec = lambda f: pl.BlockSpec((f, 1), lambda b, j: (0, 0))
    mat = lambda a, b: pl.BlockSpec((a, b), lambda i, j: (0, 0))
    tv = lambda key: mats[key].reshape(-1, 1)
    return pl.pallas_call(
        functools.partial(_sa_body, N, sblk),
        grid=(B, nblk),
        in_specs=[
            pl.BlockSpec((1, dp, N), lambda b, j: (b, 0, 0)),
            pl.BlockSpec((1, 1, NK * sblk),
                         lambda b, j: (b * nblk + j, 0, 0)),
            pl.BlockSpec((1, 8, sblk), lambda b, j: (b, 0, j)),
            mat(f1, dp), mat(f1, 8), vec(f1), vec(f1), vec(f1),
            mat(f2, f1), vec(f2), vec(f2), vec(f2),
            mat(f3, f2), vec(f3), vec(f3), vec(f3),
        ],
        out_specs=pl.BlockSpec((1, f3, sblk), lambda b, j: (b, 0, j)),
        out_shape=jax.ShapeDtypeStruct((B, f3, S), jnp.float32),
    )(tabT, idxF.reshape(B * nblk, 1, NK * sblk),
      cenT, w1pT, wr8T, tv("b1"), tv("s1"), tv("t1"),
      mats["w2"].T, tv("b2"), tv("s2"), tv("t2"),
      mats["w3"].T, tv("b3"), tv("s3"), tv("t3"))


def _sa3_body(cb, npts, *refs):
    (rows_ref, cen_ref, w1a_ref, w1b_ref, b1_ref, s1_ref, t1_ref,
     w2_ref, b2_ref, s2_ref, t2_ref, w3_ref, b3_ref, s3_ref, t3_ref,
     out_ref) = refs
    h = (jnp.dot(rows_ref[...], w1a_ref[...],
                 preferred_element_type=jnp.float32)
         + jnp.dot(cen_ref[...], w1b_ref[...],
                   preferred_element_type=jnp.float32))
    h = jnp.maximum(h + b1_ref[...], 0.0) * s1_ref[...] + t1_ref[...]
    h = jnp.dot(h, w2_ref[...], preferred_element_type=jnp.float32)
    h = jnp.maximum(h + b2_ref[...], 0.0) * s2_ref[...] + t2_ref[...]
    h = jnp.dot(h, w3_ref[...], preferred_element_type=jnp.float32)
    h = jnp.maximum(h + b3_ref[...], 0.0) * s3_ref[...] + t3_ref[...]
    f3 = h.shape[1]
    out_ref[...] = jnp.max(jnp.reshape(h, (cb, npts, f3)), axis=1)


def _sa3(rows, cen, w1a, w1b, mats, npts, cb=8):
    SB = rows.shape[0]
    B = SB // npts
    fin = rows.shape[1]
    f1 = w1a.shape[1]
    f2 = mats["w2"].shape[1]
    f3 = mats["w3"].shape[1]
    vec = lambda f: pl.BlockSpec((1, f), lambda i: (0, 0))
    mat = lambda a, b: pl.BlockSpec((a, b), lambda i: (0, 0))
    body = lambda *refs: _sa3_body(cb, npts, *refs)
    return pl.pallas_call(
        body,
        grid=(B // cb,),
        in_specs=[
            pl.BlockSpec((cb * npts, fin), lambda i: (i, 0)),
            pl.BlockSpec((cb * npts, 8), lambda i: (i, 0)),
            mat(fin, f1), mat(8, f1), vec(f1), vec(f1), vec(f1),
            mat(f1, f2), vec(f2), vec(f2), vec(f2),
            mat(f2, f3), vec(f3), vec(f3), vec(f3),
        ],
        out_specs=pl.BlockSpec((cb, f3), lambda i: (i, 0)),
        out_shape=jax.ShapeDtypeStruct((B, f3), jnp.float32),
    )(rows, cen, w1a, w1b, mats["b1"], mats["s1"], mats["t1"],
      mats["w2"], mats["b2"], mats["s2"], mats["t2"],
      mats["w3"], mats["b3"], mats["s3"], mats["t3"])


def _head_body(nseq, nstep, *refs):
    (g_ref, w1_ref, b1_ref, w2_ref, b2_ref, w3_ref, b3_ref,
     wih_ref, bih_ref, whh_ref, bhh_ref, w4_ref, b4_ref, w5_ref, b5_ref,
     out_ref) = refs
    h = jnp.maximum(jnp.dot(g_ref[...], w1_ref[...],
                            preferred_element_type=jnp.float32)
                    + b1_ref[...], 0.0)
    h = jnp.maximum(jnp.dot(h, w2_ref[...],
                            preferred_element_type=jnp.float32)
                    + b2_ref[...], 0.0)
    h = jnp.dot(h, w3_ref[...], preferred_element_type=jnp.float32) \
        + b3_ref[...]
    hid = h.shape[1]
    hs = jnp.zeros((nseq, hid), jnp.float32)
    for t in range(nstep):
        xt = h[t * nseq:(t + 1) * nseq, :]
        gi = jnp.dot(xt, wih_ref[...],
                     preferred_element_type=jnp.float32) + bih_ref[...]
        gh = jnp.dot(hs, whh_ref[...],
                     preferred_element_type=jnp.float32) + bhh_ref[...]
        i_r = gi[:, 0 * hid:1 * hid]
        i_z = gi[:, 1 * hid:2 * hid]
        i_n = gi[:, 2 * hid:3 * hid]
        h_r = gh[:, 0 * hid:1 * hid]
        h_z = gh[:, 1 * hid:2 * hid]
        h_n = gh[:, 2 * hid:3 * hid]
        rg = jax.nn.sigmoid(i_r + h_r)
        zg = jax.nn.sigmoid(i_z + h_z)
        ng = jnp.tanh(i_n + rg * h_n)
        hs = (1.0 - zg) * ng + zg * hs
    h = jnp.maximum(jnp.dot(hs, w4_ref[...],
                            preferred_element_type=jnp.float32)
                    + b4_ref[...], 0.0)
    out_ref[...] = jnp.dot(h, w5_ref[...],
                           preferred_element_type=jnp.float32) + b5_ref[...]


def _head(g, p, nseq, nstep):
    body = lambda *refs: _head_body(nseq, nstep, *refs)
    zdim = p["lin5"]["W"].shape[1]
    return pl.pallas_call(
        body,
        out_shape=jax.ShapeDtypeStruct((nseq, zdim), jnp.float32),
    )(g, p["lin1"]["W"], p["lin1"]["b"].reshape(1, -1),
      p["lin2"]["W"], p["lin2"]["b"].reshape(1, -1),
      p["lin3"]["W"], p["lin3"]["b"].reshape(1, -1),
      p["gru"]["W_ih"], p["gru"]["b_ih"].reshape(1, -1),
      p["gru"]["W_hh"], p["gru"]["b_hh"].reshape(1, -1),
      p["lin4"]["W"], p["lin4"]["b"].reshape(1, -1),
      p["lin5"]["W"], p["lin5"]["b"].reshape(1, -1))


_BN_SCALE = float((1.0 + 1e-5) ** -0.5)


def _sa_params(layers, fx, dp):
    w1 = layers[0]["W"]
    f1 = w1.shape[1]
    w1p = jnp.zeros((dp, f1), jnp.float32)
    w1p = w1p.at[:fx].set(w1[:fx]).at[fx:fx + 3].set(w1[fx:fx + 3])
    wr8 = jnp.zeros((8, f1), jnp.float32).at[:3].set(w1[fx:fx + 3])
    mats = {"b1": layers[0]["b"].reshape(1, -1),
            "s1": (layers[0]["gamma"] * _BN_SCALE).reshape(1, -1),
            "t1": layers[0]["beta"].reshape(1, -1)}
    for i, nm in ((1, "2"), (2, "3")):
        mats["w" + nm] = layers[i]["W"]
        mats["b" + nm] = layers[i]["b"].reshape(1, -1)
        mats["s" + nm] = (layers[i]["gamma"] * _BN_SCALE).reshape(1, -1)
        mats["t" + nm] = layers[i]["beta"].reshape(1, -1)
    return w1p, wr8, mats


def _cen8(cx, cy, cz):
    B, S = cx.shape
    c = jnp.stack([cx, cy, cz], axis=-1).reshape(B * S, 3)
    return jnp.concatenate([c, jnp.zeros((B * S, 5), jnp.float32)], axis=1)


def _expand_pts(p):
    return jnp.broadcast_to(p[:, :, None], (*p.shape, 8))


def kernel(x, pos, batch, params):
    B = 64
    N = 1024
    S1, S2 = 512, 128
    pr = pos.reshape(B, N, 3)
    px, py, pz = pr[:, :, 0], pr[:, :, 1], pr[:, :, 2]

    c1x, c1y, c1z = _fps(px, py, pz, S1)
    idxF1 = _nbr(_expand_pts(px), _expand_pts(py), _expand_pts(pz),
                 c1x.reshape(B, 1, S1), c1y.reshape(B, 1, S1),
                 c1z.reshape(B, 1, S1), 0.2 * 0.2, nblk=4)
    xT = x.reshape(B, N, 9).transpose(0, 2, 1)
    pT = pr.transpose(0, 2, 1)
    tabT1 = jnp.concatenate(
        [xT, pT, jnp.zeros((B, 4, N), jnp.float32)], axis=1)
    cenT1 = jnp.concatenate(
        [jnp.stack([c1x, c1y, c1z], axis=1),
         jnp.zeros((B, 5, S1), jnp.float32)], axis=1)
    w1p, wr8, mats = _sa_params(params["sa1"], 9, 16)
    h1T = _sa_fused(tabT1, idxF1, cenT1, w1p.T, wr8.T, mats, S1,
                    nblk=4)

    c2x, c2y, c2z = _fps(c1x, c1y, c1z, S2)
    idxF2 = _nbr(_expand_pts(c1x), _expand_pts(c1y), _expand_pts(c1z),
                 c2x.reshape(B, 1, S2), c2y.reshape(B, 1, S2),
                 c2z.reshape(B, 1, S2), 0.4 * 0.4, nblk=1)
    tabT2 = jnp.concatenate(
        [h1T, cenT1[:, :3], jnp.zeros((B, 5, S1), jnp.float32)],
        axis=1)
    cenT2 = jnp.concatenate(
        [jnp.stack([c2x, c2y, c2z], axis=1),
         jnp.zeros((B, 5, S2), jnp.float32)], axis=1)
    w1p2, wr82, mats2 = _sa_params(params["sa2"], 128, 136)
    h2T = _sa_fused(tabT2, idxF2, cenT2, w1p2.T, wr82.T, mats2, S2,
                    nblk=1)
    h2 = h2T.transpose(0, 2, 1).reshape(B * S2, -1)
    cen2 = _cen8(c2x, c2y, c2z)

    w1 = params["sa3"][0]["W"]
    w1a = w1[:256]
    w1b = jnp.zeros((8, 256), jnp.float32).at[:3].set(w1[256:259])
    mats3 = {"b1": params["sa3"][0]["b"].reshape(1, -1),
             "s1": (params["sa3"][0]["gamma"] * _BN_SCALE).reshape(1, -1),
             "t1": params["sa3"][0]["beta"].reshape(1, -1)}
    for i, nm in ((1, "2"), (2, "3")):
        mats3["w" + nm] = params["sa3"][i]["W"]
        mats3["b" + nm] = params["sa3"][i]["b"].reshape(1, -1)
        mats3["s" + nm] = (params["sa3"][i]["gamma"] * _BN_SCALE).reshape(1, -1)
        mats3["t" + nm] = params["sa3"][i]["beta"].reshape(1, -1)
    g = _sa3(h2, cen2, w1a, w1b, mats3, npts=S2)

    nseq, nstep = B // 4, 4
    g_r = g.reshape(nseq, nstep, -1).transpose(1, 0, 2).reshape(B, -1)
    return _head(g_r, params, nseq, nstep)
